# Initial kernel scaffold; baseline (speedup 1.0000x reference)
import jax
import jax.numpy as jnp
from jax.experimental import pallas as pl


def kernel(batch, x, edge_index, W1_0, b1_0, W2_0, b2_0, gamma_0, beta_0, W1s, b1s, W2s, b2s, gammas, betas):
    raise NotImplementedError("write your pallas kernel here")


# speedup vs baseline: 2.9940x; 2.9940x over previous
import functools

import jax
import jax.numpy as jnp
from jax import lax
from jax.experimental import pallas as pl
from jax.experimental.pallas import tpu as pltpu
from jax.experimental.pallas import tpu_sc as plsc

_N = 10000
_E = 160000
_G = 64
_D = 512
_CW = 128
_BE = 128
_NBE = 80
_EPAD = 16 * _NBE * _BE
_SPROWS = 10240
_BN = 1000
_NI = _N // _BN
_EPS = 1e-5


def _sc_segment_sum(h_all, src2, dst2, C):
    cpc = C // 2
    mesh = plsc.VectorSubcoreMesh(core_axis_name="c", subcore_axis_name="s")

    def body(h_hbm, src_hbm, dst_hbm, agg_hbm,
             src_v, dst_v, st_a, zz, acc_s):
        core = lax.axis_index("c")
        tile = lax.axis_index("s")
        pltpu.sync_copy(src_hbm.at[tile], src_v)
        pltpu.sync_copy(dst_hbm.at[tile], dst_v)
        zvec = jnp.zeros((16,), jnp.float32)

        def zrow(r, c_):
            for kk in range(_CW // 16):
                zz[r, pl.ds(kk * 16, 16)] = zvec
            return c_
        lax.fori_loop(0, 32, zrow, 0)

        for j in range(cpc):
            chunk = core * cpc + j
            def zs(i, c_):
                pltpu.sync_copy(zz, acc_s.at[pl.ds(tile * 640 + i * 32, 32)])
                return c_
            lax.fori_loop(0, 20, zs, 0)
            off = core * (cpc * _N) if j == 0 else _N
            offv = jnp.full((16,), off, jnp.int32)

            def addrow(r, c_):
                for kk in range(_BE // 16):
                    src_v[r, pl.ds(kk * 16, 16)] = (
                        src_v[r, pl.ds(kk * 16, 16)] + offv)
                return c_
            lax.fori_loop(0, _NBE, addrow, 0)
            plsc.subcore_barrier()

            def eb(b, c_):
                pltpu.sync_copy(h_hbm.at[src_v.at[b]], st_a)
                pltpu.sync_copy(st_a, acc_s.at[dst_v.at[b]], add=True)
                return c_
            lax.fori_loop(0, _NBE, eb, 0)
            plsc.subcore_barrier()
            pltpu.sync_copy(acc_s.at[pl.ds(tile * 624, 624)],
                            agg_hbm.at[pl.ds(chunk * _N + tile * 624, 624)])

            @pl.when(tile == 15)
            def _():
                pltpu.sync_copy(acc_s.at[pl.ds(9984, 16)],
                                agg_hbm.at[pl.ds(chunk * _N + 9984, 16)])
            plsc.subcore_barrier()

    f = pl.kernel(
        body,
        out_type=jax.ShapeDtypeStruct((C * _N, _CW), jnp.float32),
        mesh=mesh,
        scratch_types=[
            pltpu.VMEM((_NBE, _BE), jnp.int32),
            pltpu.VMEM((_NBE, _BE), jnp.int32),
            pltpu.VMEM((_BE, _CW), jnp.float32),
            pltpu.VMEM((32, _CW), jnp.float32),
            pltpu.VMEM_SHARED((_SPROWS, _CW), jnp.float32),
        ],
    )
    return f(h_all, src2, dst2)


def _mlp(agg_all, h_all, W1, b1, W2, b2, C):
    def body(agg_ref, h_ref, w1_ref, b1_ref, w2_ref, b2_ref,
             out_ref, st_ref, acc_ref):
        i = pl.program_id(0)
        c = pl.program_id(1)
        x = agg_ref[...] + h_ref[...]
        part = jnp.dot(x, w1_ref[...], preferred_element_type=jnp.float32)

        @pl.when(c == 0)
        def _():
            acc_ref[...] = part

        @pl.when(c != 0)
        def _():
            acc_ref[...] = acc_ref[...] + part

        @pl.when(c == C - 1)
        def _():
            t = jnp.maximum(acc_ref[...] + b1_ref[...], 0.0)
            o = (jnp.dot(t, w2_ref[...], preferred_element_type=jnp.float32)
                 + b2_ref[...])
            out_ref[...] = o
            st = jnp.concatenate(
                [jnp.sum(o, axis=0, keepdims=True),
                 jnp.sum(o * o, axis=0, keepdims=True)], axis=0)

            @pl.when(i == 0)
            def _():
                st_ref[...] = st

            @pl.when(i != 0)
            def _():
                st_ref[...] = st_ref[...] + st

    return pl.pallas_call(
        body,
        grid=(_NI, C),
        in_specs=[
            pl.BlockSpec((_BN, _CW), lambda i, c: (c * _NI + i, 0)),
            pl.BlockSpec((_BN, _CW), lambda i, c: (c * _NI + i, 0)),
            pl.BlockSpec((_CW, _D), lambda i, c: (c, 0)),
            pl.BlockSpec((1, _D), lambda i, c: (0, 0)),
            pl.BlockSpec((_D, _D), lambda i, c: (0, 0)),
            pl.BlockSpec((1, _D), lambda i, c: (0, 0)),
        ],
        out_specs=[
            pl.BlockSpec((_BN, _D), lambda i, c: (i, 0)),
            pl.BlockSpec((2, _D), lambda i, c: (0, 0)),
        ],
        out_shape=[
            jax.ShapeDtypeStruct((_N, _D), jnp.float32),
            jax.ShapeDtypeStruct((2, _D), jnp.float32),
        ],
        scratch_shapes=[pltpu.VMEM((_BN, _D), jnp.float32)],
    )(agg_all, h_all, W1, b1.reshape(1, _D), W2, b2.reshape(1, _D))


def _bn_relu_chunked(out, st, gamma, beta):
    def body(o_ref, st_ref, g_ref, b_ref, h_ref):
        mean = st_ref[0, :] / _N
        var = st_ref[1, :] / _N - mean * mean
        scale = g_ref[0, :] * lax.rsqrt(var + _EPS)
        shift = b_ref[0, :] - mean * scale
        y = o_ref[...] * scale[None, :] + shift[None, :]
        h_ref[...] = jnp.maximum(y, 0.0)

    return pl.pallas_call(
        body,
        grid=(_NI, 4),
        in_specs=[
            pl.BlockSpec((_BN, _CW), lambda i, c: (i, c)),
            pl.BlockSpec((2, _CW), lambda i, c: (0, c)),
            pl.BlockSpec((1, _CW), lambda i, c: (0, c)),
            pl.BlockSpec((1, _CW), lambda i, c: (0, c)),
        ],
        out_specs=pl.BlockSpec((_BN, _CW), lambda i, c: (c * _NI + i, 0)),
        out_shape=jax.ShapeDtypeStruct((4 * _N, _CW), jnp.float32),
    )(out, st, gamma.reshape(1, _D), beta.reshape(1, _D))


def _bn_pool_final(out, st, gamma, beta, batch3):
    def body(o_ref, st_ref, g_ref, b_ref, bt_ref, h_ref, xp_ref):
        i = pl.program_id(0)
        mean = st_ref[0, :] / _N
        var = st_ref[1, :] / _N - mean * mean
        scale = g_ref[0, :] * lax.rsqrt(var + _EPS)
        shift = b_ref[0, :] - mean * scale
        y = o_ref[...] * scale[None, :] + shift[None, :]
        h_ref[...] = y
        bb = bt_ref[0, 0, :]
        oh = (lax.broadcasted_iota(jnp.int32, (_G, _BN), 0)
              == bb[None, :]).astype(jnp.float32)
        xp = jnp.dot(oh, y, preferred_element_type=jnp.float32)

        @pl.when(i == 0)
        def _():
            xp_ref[...] = xp

        @pl.when(i != 0)
        def _():
            xp_ref[...] = xp_ref[...] + xp

    return pl.pallas_call(
        body,
        grid=(_NI,),
        in_specs=[
            pl.BlockSpec((_BN, _D), lambda i: (i, 0)),
            pl.BlockSpec((2, _D), lambda i: (0, 0)),
            pl.BlockSpec((1, _D), lambda i: (0, 0)),
            pl.BlockSpec((1, _D), lambda i: (0, 0)),
            pl.BlockSpec((1, 1, _BN), lambda i: (i, 0, 0)),
        ],
        out_specs=[
            pl.BlockSpec((_BN, _D), lambda i: (i, 0)),
            pl.BlockSpec((_G, _D), lambda i: (0, 0)),
        ],
        out_shape=[
            jax.ShapeDtypeStruct((_N, _D), jnp.float32),
            jax.ShapeDtypeStruct((_G, _D), jnp.float32),
        ],
    )(out, st, gamma.reshape(1, _D), beta.reshape(1, _D), batch3)


def kernel(batch, x, edge_index, W1_0, b1_0, W2_0, b2_0, gamma_0, beta_0,
           W1s, b1s, W2s, b2s, gammas, betas):
    src = edge_index[0]
    dst = edge_index[1]
    pad = _EPAD - _E
    src_p = jnp.concatenate([src, jnp.zeros((pad,), jnp.int32)])
    dst_p = jnp.concatenate([dst, jnp.full((pad,), _N, jnp.int32)])
    order = jnp.argsort(dst_p)
    src_s = src_p[order]
    dst_s = dst_p[order]
    nbat = 16 * _NBE
    src2 = src_s.reshape(_BE, nbat).T.reshape(16, _NBE, _BE)
    dst2 = dst_s.reshape(_BE, nbat).T.reshape(16, _NBE, _BE)
    x_all = x.reshape(_N, 2, _CW).transpose(1, 0, 2).reshape(2 * _N, _CW)
    batch3 = batch.reshape(_NI, 1, _BN)

    h_all = x_all
    C = 2
    h = None
    xpool = None
    for layer in range(5):
        if layer == 0:
            W1, b1, W2, b2 = W1_0, b1_0, W2_0, b2_0
            g, bt = gamma_0, beta_0
        else:
            W1, b1 = W1s[layer - 1], b1s[layer - 1]
            W2, b2 = W2s[layer - 1], b2s[layer - 1]
            g, bt = gammas[layer - 1], betas[layer - 1]
        agg_all = _sc_segment_sum(h_all, src2, dst2, C)
        out, st = _mlp(agg_all, h_all, W1, b1, W2, b2, C)
        if layer < 4:
            h_all = _bn_relu_chunked(out, st, g, bt)
            C = 4
        else:
            h, xpool = _bn_pool_final(out, st, g, bt, batch3)
    return (xpool, h)
